# Initial kernel scaffold; baseline (speedup 1.0000x reference)
import functools

import jax
import jax.numpy as jnp
from jax import lax
from jax.experimental import pallas as pl
from jax.experimental.pallas import tpu as pltpu
from jax.experimental.pallas import tpu_sc as plsc

N_USERS = 100000
N_MOVIES = 100000
F = 128
B = 16384
NW = 32
BPW = B // NW
CHUNK = 128
NCHUNK = BPW // CHUNK
L = 16


def _sc_body(user_hbm, movie_hbm, uf_hbm, mf_hbm, ub_hbm, mb_hbm, out_hbm,
             uidx_v, midx_v, ubias_v, mbias_v, ubuf, mbuf, out_v,
             sem_b, sem_f):
    wid = lax.axis_index("s") * 2 + lax.axis_index("c")

    pltpu.sync_copy(user_hbm.at[wid], uidx_v)
    pltpu.sync_copy(movie_hbm.at[wid], midx_v)

    bias_copies = []
    for c in range(NCHUNK):
        bias_copies.append(
            pltpu.async_copy(ub_hbm.at[uidx_v.at[c]], ubias_v.at[c], sem_b))
        bias_copies.append(
            pltpu.async_copy(mb_hbm.at[midx_v.at[c]], mbias_v.at[c], sem_b))

    def start_chunk(c):
        p = c % 2
        cu = pltpu.async_copy(uf_hbm.at[uidx_v.at[c]], ubuf.at[p], sem_f)
        cm = pltpu.async_copy(mf_hbm.at[midx_v.at[c]], mbuf.at[p], sem_f)
        return (cu, cm)

    inflight = {0: start_chunk(0)}
    for cp in bias_copies:
        cp.wait()

    rows_iota = lax.iota(jnp.int32, L)

    for c in range(NCHUNK):
        if c + 1 < NCHUNK:
            inflight[c + 1] = start_chunk(c + 1)
        cu, cm = inflight.pop(c)
        cu.wait()
        cm.wait()
        p = c % 2
        ub_c = ubuf.at[p]
        mb_c = mbuf.at[p]
        for g in range(CHUNK // L):
            rows = rows_iota + (g * L)

            def body(f, acc):
                fvec = jnp.zeros((L,), jnp.int32) + f
                u = plsc.load_gather(ub_c, [rows, fvec])
                m = plsc.load_gather(mb_c, [rows, fvec])
                return acc + u * m

            acc = lax.fori_loop(0, F, body, jnp.zeros((L,), jnp.float32))
            acc = acc + ubias_v[c, pl.ds(g * L, L)] + mbias_v[c, pl.ds(g * L, L)]
            out_v[c, pl.ds(g * L, L)] = acc

    pltpu.sync_copy(out_v, out_hbm.at[wid])


@jax.jit
def _run(user_r, movie_r, uf, mf, ub, mb):
    mesh = plsc.VectorSubcoreMesh(core_axis_name="c", subcore_axis_name="s")
    kfn = pl.kernel(
        _sc_body,
        out_type=jax.ShapeDtypeStruct((NW, NCHUNK, CHUNK), jnp.float32),
        mesh=mesh,
        scratch_types=[
            pltpu.VMEM((NCHUNK, CHUNK), jnp.int32),
            pltpu.VMEM((NCHUNK, CHUNK), jnp.int32),
            pltpu.VMEM((NCHUNK, CHUNK), jnp.float32),
            pltpu.VMEM((NCHUNK, CHUNK), jnp.float32),
            pltpu.VMEM((2, CHUNK, F), jnp.float32),
            pltpu.VMEM((2, CHUNK, F), jnp.float32),
            pltpu.VMEM((NCHUNK, CHUNK), jnp.float32),
            pltpu.SemaphoreType.DMA,
            pltpu.SemaphoreType.DMA,
        ],
    )
    return kfn(user_r, movie_r, uf, mf, ub, mb)


def kernel(user, movie, user_factors, movie_factors, user_biases, movie_biases):
    user_r = user.astype(jnp.int32).reshape(NW, NCHUNK, CHUNK)
    movie_r = movie.astype(jnp.int32).reshape(NW, NCHUNK, CHUNK)
    ub = user_biases.reshape(N_USERS)
    mb = movie_biases.reshape(N_MOVIES)
    out = _run(user_r, movie_r, user_factors, movie_factors, ub, mb)
    return out.reshape(B)


# speedup vs baseline: 1.3634x; 1.3634x over previous
import functools

import jax
import jax.numpy as jnp
from jax import lax
from jax.experimental import pallas as pl
from jax.experimental.pallas import tpu as pltpu
from jax.experimental.pallas import tpu_sc as plsc

N_USERS = 100000
N_MOVIES = 100000
F = 128
B = 16384
NW = 32
BPW = B // NW
CHUNK = 128
NCHUNK = BPW // CHUNK
L = 16


def _sc_body(user_hbm, movie_hbm, uf_hbm, mf_hbm, ub_hbm, mb_hbm, out_hbm,
             uidx_v, midx_v, ubias_v, mbias_v, ubuf0, ubuf1, mbuf0, mbuf1,
             out_v, sem_b, sem_f):
    ubufs = (ubuf0, ubuf1)
    mbufs = (mbuf0, mbuf1)
    wid = lax.axis_index("s") * 2 + lax.axis_index("c")

    pltpu.sync_copy(user_hbm.at[wid], uidx_v)
    pltpu.sync_copy(movie_hbm.at[wid], midx_v)

    bias_copies = []
    for c in range(NCHUNK):
        bias_copies.append(
            pltpu.async_copy(ub_hbm.at[uidx_v.at[c]], ubias_v.at[c], sem_b))
        bias_copies.append(
            pltpu.async_copy(mb_hbm.at[midx_v.at[c]], mbias_v.at[c], sem_b))

    def start_chunk(c):
        p = c % 2
        cu = pltpu.async_copy(uf_hbm.at[uidx_v.at[c]], ubufs[p], sem_f)
        cm = pltpu.async_copy(mf_hbm.at[midx_v.at[c]], mbufs[p], sem_f)
        return (cu, cm)

    inflight = {0: start_chunk(0)}
    for cp in bias_copies:
        cp.wait()

    lane_masks = [jnp.arange(L) == r for r in range(L)]

    for c in range(NCHUNK):
        if c + 1 < NCHUNK:
            inflight[c + 1] = start_chunk(c + 1)
        cu, cm = inflight.pop(c)
        cu.wait()
        cm.wait()
        p = c % 2
        ub_c = ubufs[p]
        mb_c = mbufs[p]

        def group_body(g, _, ub_c=ub_c, mb_c=mb_c, c=c):
            acc = jnp.zeros((L,), jnp.float32)
            for r in range(L):
                row = g * L + r
                prods = [ub_c[row, pl.ds(k * L, L)] * mb_c[row, pl.ds(k * L, L)]
                         for k in range(F // L)]
                while len(prods) > 1:
                    prods = [prods[i] + prods[i + 1]
                             for i in range(0, len(prods), 2)]
                acc = jnp.where(lane_masks[r], jnp.sum(prods[0]), acc)
            acc = (acc + ubias_v[c, pl.ds(g * L, L)]
                   + mbias_v[c, pl.ds(g * L, L)])
            out_v[c, pl.ds(g * L, L)] = acc
            return 0

        lax.fori_loop(0, CHUNK // L, group_body, 0)

    pltpu.sync_copy(out_v, out_hbm.at[wid])


@jax.jit
def _run(user_r, movie_r, uf, mf, ub, mb):
    mesh = plsc.VectorSubcoreMesh(core_axis_name="c", subcore_axis_name="s")
    kfn = pl.kernel(
        _sc_body,
        out_type=jax.ShapeDtypeStruct((NW, NCHUNK, CHUNK), jnp.float32),
        mesh=mesh,
        compiler_params=pltpu.CompilerParams(needs_layout_passes=False),
        scratch_types=[
            pltpu.VMEM((NCHUNK, CHUNK), jnp.int32),
            pltpu.VMEM((NCHUNK, CHUNK), jnp.int32),
            pltpu.VMEM((NCHUNK, CHUNK), jnp.float32),
            pltpu.VMEM((NCHUNK, CHUNK), jnp.float32),
            pltpu.VMEM((CHUNK, F), jnp.float32),
            pltpu.VMEM((CHUNK, F), jnp.float32),
            pltpu.VMEM((CHUNK, F), jnp.float32),
            pltpu.VMEM((CHUNK, F), jnp.float32),
            pltpu.VMEM((NCHUNK, CHUNK), jnp.float32),
            pltpu.SemaphoreType.DMA,
            pltpu.SemaphoreType.DMA,
        ],
    )
    return kfn(user_r, movie_r, uf, mf, ub, mb)


def kernel(user, movie, user_factors, movie_factors, user_biases, movie_biases):
    user_r = user.astype(jnp.int32).reshape(NW, NCHUNK, CHUNK)
    movie_r = movie.astype(jnp.int32).reshape(NW, NCHUNK, CHUNK)
    ub = user_biases.reshape(N_USERS)
    mb = movie_biases.reshape(N_MOVIES)
    out = _run(user_r, movie_r, user_factors, movie_factors, ub, mb)
    return out.reshape(B)
